# baseline (speedup 1.0000x reference)
import functools

import jax
import jax.numpy as jnp
from jax import lax
from jax.experimental import pallas as pl
from jax.experimental.pallas import tpu as pltpu
from jax.experimental.pallas import tpu_sc as plsc

MAXK = 8
_NEG = float(jnp.finfo(jnp.float32).min)

_RSEL = 256


def _select_body(scores_ref, mask_ref, bidx_ref, flat_ref, picked_ref):
    s = scores_ref[...]
    m = mask_ref[...]
    R, L = s.shape
    ms = jnp.where(m != 0, s, _NEG)
    iota = lax.broadcasted_iota(jnp.int32, (R, L), 1)
    idx_cols, picked_cols = [], []
    for _ in range(MAXK):
        mx = jnp.max(ms, axis=1, keepdims=True)
        idx = jnp.argmax(ms, axis=1, keepdims=True).astype(jnp.int32)
        idx_cols.append(idx)
        picked_cols.append(jnp.where(mx > _NEG, 1, 0))
        ms = jnp.where(iota == idx, -jnp.inf, ms)
    idxs = jnp.concatenate(idx_cols, axis=1)
    picked = jnp.concatenate(picked_cols, axis=1)
    keys = jnp.where(picked > 0, idxs, L)
    lane8 = lax.broadcasted_iota(jnp.int32, (R, MAXK), 1)
    rank = jnp.zeros((R, MAXK), jnp.int32)
    for l in range(MAXK):
        kl = keys[:, l:l + 1]
        rank = rank + jnp.where(kl < keys, 1, 0)
        rank = rank + jnp.where((kl == keys) & (lane8 > l), 1, 0)
    skey = jnp.zeros((R, MAXK), jnp.int32)
    spick = jnp.zeros((R, MAXK), jnp.float32)
    for j in range(MAXK):
        sel = rank[:, j:j + 1] == lane8
        skey = jnp.where(sel, keys[:, j:j + 1], skey)
        spick = jnp.where(sel & (picked[:, j:j + 1] > 0), 1.0, spick)
    safe = jnp.where(spick > 0.0, skey, 0)
    flat_ref[...] = bidx_ref[...] * L + safe
    picked_ref[...] = spick


def _select(rank_scores, mask8, bidx, L):
    n = rank_scores.shape[0]
    grid = n // _RSEL
    return pl.pallas_call(
        _select_body,
        grid=(grid,),
        in_specs=[
            pl.BlockSpec((_RSEL, L), lambda i: (i, 0)),
            pl.BlockSpec((_RSEL, L), lambda i: (i, 0)),
            pl.BlockSpec((_RSEL, 1), lambda i: (i, 0)),
        ],
        out_specs=[
            pl.BlockSpec((_RSEL, MAXK), lambda i: (i, 0)),
            pl.BlockSpec((_RSEL, MAXK), lambda i: (i, 0)),
        ],
        out_shape=[
            jax.ShapeDtypeStruct((n, MAXK), jnp.int32),
            jax.ShapeDtypeStruct((n, MAXK), jnp.float32),
        ],
    )(rank_scores, mask8, bidx)


_NCORES = 2
_NSUB = 16
_NW = _NCORES * _NSUB
_ICHUNK = 128


def _sc_gather(k_flat, flat_idx):
    nrows = flat_idx.shape[0] * flat_idx.shape[1] * flat_idx.shape[2]
    D = k_flat.shape[1]
    bpw = nrows // _NW
    nchunk = bpw // _ICHUNK
    mesh = plsc.VectorSubcoreMesh(core_axis_name="c", subcore_axis_name="s")

    @functools.partial(
        pl.kernel,
        mesh=mesh,
        out_type=jax.ShapeDtypeStruct((nrows, D), jnp.float32),
        scratch_types=[
            pltpu.VMEM((nchunk, _ICHUNK), jnp.int32),
            pltpu.VMEM((bpw, D), jnp.float32),
            pltpu.SemaphoreType.DMA,
        ],
    )
    def gather_k(idx_hbm, table_hbm, out_hbm, idx_v, rows_v, sem):
        wid = lax.axis_index("s") * _NCORES + lax.axis_index("c")
        pltpu.sync_copy(idx_hbm.at[wid], idx_v)
        copies = []
        for j in range(nchunk):
            copies.append(pltpu.async_copy(
                table_hbm.at[idx_v.at[j]],
                rows_v.at[pl.ds(j * _ICHUNK, _ICHUNK)], sem))
        for c in copies:
            c.wait()
        pltpu.sync_copy(rows_v, out_hbm.at[pl.ds(wid * bpw, bpw)])

    return gather_k(flat_idx, k_flat)


_RMLP = 256


def _mlp_body(q_ref, p_ref, pick_ref, cnt_ref, w1q_ref, w1p_ref, w1c_ref,
              b1_ref, w2_ref, b2_ref, out_ref):
    q = q_ref[...]
    p = p_ref[...]
    R, d = q.shape
    pick = pick_ref[...]
    srow = lax.broadcasted_iota(jnp.int32, (MAXK, MAXK * d), 0)
    scol = lax.broadcasted_iota(jnp.int32, (MAXK, MAXK * d), 1) // d
    expand = jnp.where(srow == scol, 1.0, 0.0)
    rank1 = (((1,), (0,)), ((), ()))
    pickrep = lax.dot_general(pick, expand, rank1,
                              preferred_element_type=jnp.float32)
    pm = p * pickrep
    logc = jnp.log1p(cnt_ref[...])
    cdims = (((1,), (1,)), ((), ()))
    h = (lax.dot_general(q, w1q_ref[...], cdims,
                         preferred_element_type=jnp.float32)
         + lax.dot_general(pm, w1p_ref[...], cdims,
                           preferred_element_type=jnp.float32)
         + lax.dot_general(logc, w1c_ref[...], rank1,
                           preferred_element_type=jnp.float32)
         + b1_ref[...])
    g = 0.5 * h * (1.0 + lax.erf(h * 0.7071067811865476))
    out_ref[...] = jnp.sum(g * w2_ref[...], axis=1, keepdims=True) + b2_ref[0, 0]


def _mlp(q, packed, picked, cnt, w1q, w1p, w1c, b1r, W2, b2r):
    n, d = q.shape
    hidden = w1q.shape[0]
    grid = n // _RMLP
    full = lambda i: (0, 0)
    return pl.pallas_call(
        _mlp_body,
        grid=(grid,),
        in_specs=[
            pl.BlockSpec((_RMLP, d), lambda i: (i, 0)),
            pl.BlockSpec((_RMLP, MAXK * d), lambda i: (i, 0)),
            pl.BlockSpec((_RMLP, MAXK), lambda i: (i, 0)),
            pl.BlockSpec((_RMLP, 1), lambda i: (i, 0)),
            pl.BlockSpec((hidden, d), full),
            pl.BlockSpec((hidden, MAXK * d), full),
            pl.BlockSpec((1, hidden), full),
            pl.BlockSpec((1, hidden), full),
            pl.BlockSpec((1, hidden), full),
            pl.BlockSpec((1, 1), full, memory_space=pltpu.SMEM),
        ],
        out_specs=pl.BlockSpec((_RMLP, 1), lambda i: (i, 0)),
        out_shape=jax.ShapeDtypeStruct((n, 1), jnp.float32),
    )(q, packed, picked, cnt, w1q, w1p, w1c, b1r, W2, b2r)


def kernel(q, k, batch_idx, mask, count, rank_scores, W1, b1, W2, b2):
    n, d = q.shape
    B, L, _ = k.shape
    hidden = W1.shape[0]

    mask8 = mask
    bidx = batch_idx.astype(jnp.int32).reshape(n, 1)
    flat_idx, picked = _select(rank_scores, mask8, bidx, L)

    k_flat = k.reshape(B * L, d)
    idx3 = flat_idx.reshape(_NW, (n * MAXK) // (_NW * _ICHUNK), _ICHUNK)
    packed = _sc_gather(k_flat, idx3)
    packed2 = packed.reshape(n, MAXK * d)

    cnt = count.astype(jnp.float32).reshape(n, 1)
    w1q = W1[:, :d]
    w1p = W1[:, d:d + MAXK * d]
    w1c = W1[:, d + MAXK * d].reshape(1, hidden)
    out = _mlp(q, packed2, picked, cnt, w1q, w1p, w1c,
               b1.reshape(1, hidden), W2, b2.reshape(1, 1))
    return out.reshape(n)


# speedup vs baseline: 1.0426x; 1.0426x over previous
import functools

import jax
import jax.numpy as jnp
from jax import lax
from jax.experimental import pallas as pl
from jax.experimental.pallas import tpu as pltpu
from jax.experimental.pallas import tpu_sc as plsc

MAXK = 8
_NEG = float(jnp.finfo(jnp.float32).min)

_RSEL = 256


def _select_body(scores_ref, mask_ref, bidx_ref, flat_ref, picked_ref):
    s = scores_ref[...]
    m = mask_ref[...]
    R, L = s.shape
    ms = jnp.where(m != 0, s, _NEG)
    iota = lax.broadcasted_iota(jnp.int32, (R, L), 1)
    idx_cols, picked_cols = [], []
    nmask = jnp.sum(jnp.where(m, 1, 0), axis=1, keepdims=True)
    for t in range(MAXK):
        idx = jnp.argmax(ms, axis=1, keepdims=True).astype(jnp.int32)
        idx_cols.append(idx)
        picked_cols.append(jnp.where(nmask > t, 1, 0))
        ms = jnp.where(iota == idx, -jnp.inf, ms)
    idxs = jnp.concatenate(idx_cols, axis=1)
    picked = jnp.concatenate(picked_cols, axis=1)
    keys = jnp.where(picked > 0, idxs, L)
    lane8 = lax.broadcasted_iota(jnp.int32, (R, MAXK), 1)
    rank = jnp.zeros((R, MAXK), jnp.int32)
    for l in range(MAXK):
        kl = keys[:, l:l + 1]
        rank = rank + jnp.where(kl < keys, 1, 0)
        rank = rank + jnp.where((kl == keys) & (lane8 > l), 1, 0)
    skey = jnp.zeros((R, MAXK), jnp.int32)
    spick = jnp.zeros((R, MAXK), jnp.float32)
    for j in range(MAXK):
        sel = rank[:, j:j + 1] == lane8
        skey = jnp.where(sel, keys[:, j:j + 1], skey)
        spick = jnp.where(sel & (picked[:, j:j + 1] > 0), 1.0, spick)
    safe = jnp.where(spick > 0.0, skey, 0)
    flat_ref[...] = bidx_ref[...] * L + safe
    picked_ref[...] = spick


def _select(rank_scores, mask8, bidx, L):
    n = rank_scores.shape[0]
    grid = n // _RSEL
    return pl.pallas_call(
        _select_body,
        grid=(grid,),
        in_specs=[
            pl.BlockSpec((_RSEL, L), lambda i: (i, 0)),
            pl.BlockSpec((_RSEL, L), lambda i: (i, 0)),
            pl.BlockSpec((_RSEL, 1), lambda i: (i, 0)),
        ],
        out_specs=[
            pl.BlockSpec((_RSEL, MAXK), lambda i: (i, 0)),
            pl.BlockSpec((_RSEL, MAXK), lambda i: (i, 0)),
        ],
        out_shape=[
            jax.ShapeDtypeStruct((n, MAXK), jnp.int32),
            jax.ShapeDtypeStruct((n, MAXK), jnp.float32),
        ],
    )(rank_scores, mask8, bidx)


_NCORES = 2
_NSUB = 16
_NW = _NCORES * _NSUB
_ICHUNK = 128


def _sc_gather(k_flat, flat_idx):
    nrows = flat_idx.shape[0] * flat_idx.shape[1] * flat_idx.shape[2]
    D = k_flat.shape[1]
    bpw = nrows // _NW
    nchunk = bpw // _ICHUNK
    mesh = plsc.VectorSubcoreMesh(core_axis_name="c", subcore_axis_name="s")

    @functools.partial(
        pl.kernel,
        mesh=mesh,
        out_type=jax.ShapeDtypeStruct((nrows, D), jnp.float32),
        scratch_types=[
            pltpu.VMEM((nchunk, _ICHUNK), jnp.int32),
            pltpu.VMEM((bpw, D), jnp.float32),
            pltpu.SemaphoreType.DMA,
        ],
    )
    def gather_k(idx_hbm, table_hbm, out_hbm, idx_v, rows_v, sem):
        wid = lax.axis_index("s") * _NCORES + lax.axis_index("c")
        pltpu.sync_copy(idx_hbm.at[wid], idx_v)
        copies = []
        for j in range(nchunk):
            copies.append(pltpu.async_copy(
                table_hbm.at[idx_v.at[j]],
                rows_v.at[pl.ds(j * _ICHUNK, _ICHUNK)], sem))
        for c in copies:
            c.wait()
        pltpu.sync_copy(rows_v, out_hbm.at[pl.ds(wid * bpw, bpw)])

    return gather_k(flat_idx, k_flat)


_RMLP = 256


def _mlp_body(q_ref, p_ref, pick_ref, cnt_ref, w1q_ref, w1p_ref, w1c_ref,
              b1_ref, w2_ref, b2_ref, out_ref):
    q = q_ref[...]
    p = p_ref[...]
    R, d = q.shape
    pick = pick_ref[...]
    srow = lax.broadcasted_iota(jnp.int32, (MAXK, MAXK * d), 0)
    scol = lax.broadcasted_iota(jnp.int32, (MAXK, MAXK * d), 1) // d
    expand = jnp.where(srow == scol, 1.0, 0.0)
    rank1 = (((1,), (0,)), ((), ()))
    pickrep = lax.dot_general(pick, expand, rank1,
                              preferred_element_type=jnp.float32)
    pm = p * pickrep
    logc = jnp.log1p(cnt_ref[...])
    cdims = (((1,), (1,)), ((), ()))
    h = (lax.dot_general(q, w1q_ref[...], cdims,
                         preferred_element_type=jnp.float32)
         + lax.dot_general(pm, w1p_ref[...], cdims,
                           preferred_element_type=jnp.float32)
         + lax.dot_general(logc, w1c_ref[...], rank1,
                           preferred_element_type=jnp.float32)
         + b1_ref[...])
    g = 0.5 * h * (1.0 + lax.erf(h * 0.7071067811865476))
    out_ref[...] = jnp.sum(g * w2_ref[...], axis=1, keepdims=True) + b2_ref[0, 0]


def _mlp(q, packed, picked, cnt, w1q, w1p, w1c, b1r, W2, b2r):
    n, d = q.shape
    hidden = w1q.shape[0]
    grid = n // _RMLP
    full = lambda i: (0, 0)
    return pl.pallas_call(
        _mlp_body,
        grid=(grid,),
        in_specs=[
            pl.BlockSpec((_RMLP, d), lambda i: (i, 0)),
            pl.BlockSpec((_RMLP, MAXK * d), lambda i: (i, 0)),
            pl.BlockSpec((_RMLP, MAXK), lambda i: (i, 0)),
            pl.BlockSpec((_RMLP, 1), lambda i: (i, 0)),
            pl.BlockSpec((hidden, d), full),
            pl.BlockSpec((hidden, MAXK * d), full),
            pl.BlockSpec((1, hidden), full),
            pl.BlockSpec((1, hidden), full),
            pl.BlockSpec((1, hidden), full),
            pl.BlockSpec((1, 1), full, memory_space=pltpu.SMEM),
        ],
        out_specs=pl.BlockSpec((_RMLP, 1), lambda i: (i, 0)),
        out_shape=jax.ShapeDtypeStruct((n, 1), jnp.float32),
    )(q, packed, picked, cnt, w1q, w1p, w1c, b1r, W2, b2r)


def kernel(q, k, batch_idx, mask, count, rank_scores, W1, b1, W2, b2):
    n, d = q.shape
    B, L, _ = k.shape
    hidden = W1.shape[0]

    mask8 = mask
    bidx = batch_idx.astype(jnp.int32).reshape(n, 1)
    flat_idx, picked = _select(rank_scores, mask8, bidx, L)

    k_flat = k.reshape(B * L, d)
    idx3 = flat_idx.reshape(_NW, (n * MAXK) // (_NW * _ICHUNK), _ICHUNK)
    packed = _sc_gather(k_flat, idx3)
    packed2 = packed.reshape(n, MAXK * d)

    cnt = count.astype(jnp.float32).reshape(n, 1)
    w1q = W1[:, :d]
    w1p = W1[:, d:d + MAXK * d]
    w1c = W1[:, d + MAXK * d].reshape(1, hidden)
    out = _mlp(q, packed2, picked, cnt, w1q, w1p, w1c,
               b1.reshape(1, hidden), W2, b2.reshape(1, 1))
    return out.reshape(n)
